# Initial kernel scaffold; baseline (speedup 1.0000x reference)
import jax
import jax.numpy as jnp
from jax.experimental import pallas as pl


def kernel(x_, y_, table, W1, b1, W2, b2):
    raise NotImplementedError("write your pallas kernel here")


# speedup vs baseline: 15.7546x; 15.7546x over previous
import functools

import jax
import jax.numpy as jnp
from jax import lax
from jax.experimental import pallas as pl
from jax.experimental.pallas import tpu as pltpu
from jax.experimental.pallas import tpu_sc as plsc

VOCAB = 100000
DIM = 128
HID = 16
CLASSES = 2
B = 4096
L = 200

NC = 2
NS = 16
NW = NC * NS
ROWS_PER_W = B // NW
CHUNK = 100
CHUNKS_PER_W = ROWS_PER_W * 2


def _proj_body(table_ref, w1_ref, b1_ref, out_ref):
    out_ref[...] = (
        jnp.dot(table_ref[...], w1_ref[...], preferred_element_type=jnp.float32)
        + b1_ref[...]
    )


def _project(table, w1, b1row):
    grid = 10
    rows = VOCAB // grid
    return pl.pallas_call(
        _proj_body,
        grid=(grid,),
        in_specs=[
            pl.BlockSpec((rows, DIM), lambda i: (i, 0)),
            pl.BlockSpec((DIM, HID), lambda i: (0, 0)),
            pl.BlockSpec((1, HID), lambda i: (0, 0)),
        ],
        out_specs=pl.BlockSpec((rows, HID), lambda i: (i, 0)),
        out_shape=jax.ShapeDtypeStruct((VOCAB, HID), jnp.float32),
    )(table, w1, b1row)


def _accum(buf):
    a0 = buf[0]
    a1 = buf[1]
    a2 = buf[2]
    a3 = buf[3]
    for j in range(4, 2 * CHUNK, 4):
        a0 = a0 + buf[j]
        a1 = a1 + buf[j + 1]
        a2 = a2 + buf[j + 2]
        a3 = a3 + buf[j + 3]
    return (a0 + a1) + (a2 + a3)


def _gather_sum(xr, p):
    mesh = plsc.VectorSubcoreMesh(core_axis_name="c", subcore_axis_name="s")

    @functools.partial(
        pl.kernel,
        out_type=jax.ShapeDtypeStruct((B, HID), jnp.float32),
        mesh=mesh,
        scratch_types=[
            pltpu.VMEM((CHUNKS_PER_W, CHUNK), jnp.int32),
            pltpu.VMEM((2 * CHUNK, HID), jnp.float32),
            pltpu.VMEM((2 * CHUNK, HID), jnp.float32),
            pltpu.VMEM((ROWS_PER_W, HID), jnp.float32),
            pltpu.SemaphoreType.DMA,
            pltpu.SemaphoreType.DMA,
            pltpu.SemaphoreType.DMA,
        ],
        compiler_params=pltpu.CompilerParams(use_tc_tiling_on_sc=False),
    )
    def body(x_hbm, p_hbm, out_hbm, idx_v, buf0, buf1, out_v, semi, sem0, sem1):
        wid = lax.axis_index("s") * NC + lax.axis_index("c")
        pltpu.async_copy(x_hbm.at[wid], idx_v, semi).wait()

        def fire_row(r, buf, sem):
            pltpu.async_copy(p_hbm.at[idx_v.at[2 * r]],
                             buf.at[pl.ds(0, CHUNK)], sem)
            pltpu.async_copy(p_hbm.at[idx_v.at[2 * r + 1]],
                             buf.at[pl.ds(CHUNK, CHUNK)], sem)

        def wait_row(buf, sem):
            pltpu.make_async_copy(p_hbm.at[pl.ds(0, 2 * CHUNK)], buf, sem).wait()

        fire_row(0, buf0, sem0)
        fire_row(1, buf1, sem1)

        def step(i, _):
            wait_row(buf0, sem0)
            acc0 = _accum(buf0)
            out_v[2 * i, :] = acc0

            @pl.when(i < ROWS_PER_W // 2 - 1)
            def _():
                fire_row(2 * i + 2, buf0, sem0)

            wait_row(buf1, sem1)
            acc1 = _accum(buf1)
            out_v[2 * i + 1, :] = acc1

            @pl.when(i < ROWS_PER_W // 2 - 1)
            def _():
                fire_row(2 * i + 3, buf1, sem1)

            return 0

        lax.fori_loop(0, ROWS_PER_W // 2, step, 0)
        pltpu.sync_copy(out_v, out_hbm.at[pl.ds(wid * ROWS_PER_W, ROWS_PER_W)])

    return body(xr, p)


def _head_body(s_ref, y_ref, w2t_ref, b2_ref, out_ref):
    h = jnp.maximum(s_ref[...] * (1.0 / L), 0.0)
    w2t = w2t_ref[...]
    l0 = jnp.sum(h * w2t[0][None, :], axis=1) + b2_ref[0, 0]
    l1 = jnp.sum(h * w2t[1][None, :], axis=1) + b2_ref[0, 1]
    m = jnp.maximum(l0, l1)
    lse = m + jnp.log(jnp.exp(l0 - m) + jnp.exp(l1 - m))
    picked = jnp.where(y_ref[...] == 0, l0, l1)
    out_ref[...] = (jnp.sum(lse - picked) * (1.0 / B)).reshape(1, 1)


def _head(s, y, w2t, b2row):
    return pl.pallas_call(
        _head_body,
        out_shape=jax.ShapeDtypeStruct((1, 1), jnp.float32),
    )(s, y, w2t, b2row)


def kernel(x_, y_, table, W1, b1, W2, b2):
    p = _project(table, W1, b1.reshape(1, HID))
    xr = x_.astype(jnp.int32).reshape(NW, CHUNKS_PER_W, CHUNK)
    s = _gather_sum(xr, p)
    out = _head(s, y_, W2.T, b2.reshape(1, CLASSES))
    return out[0, 0]


# speedup vs baseline: 19.2185x; 1.2199x over previous
import functools

import jax
import jax.numpy as jnp
from jax import lax
from jax.experimental import pallas as pl
from jax.experimental.pallas import tpu as pltpu
from jax.experimental.pallas import tpu_sc as plsc

VOCAB = 100000
DIM = 128
HID = 16
CLASSES = 2
B = 4096
L = 200

NC = 2
NS = 16
NW = NC * NS
ROWS_PER_W = B // NW
CHUNK = 100
CHUNKS_PER_W = ROWS_PER_W * 2


VOCAB_PAD = 100352
PACK_ROWS = 12544
_GRID = 14
_BROWS = 896


def _proj_body(*refs):
    t_refs = refs[:8]
    w1_ref, b1_ref, out_ref = refs[8:]
    w1 = w1_ref[...]
    b1 = b1_ref[...]
    parts = [
        jnp.dot(t[...], w1, preferred_element_type=jnp.float32) + b1
        for t in t_refs
    ]
    out_ref[...] = jnp.concatenate(parts, axis=1)


def _project(table, w1, b1row):
    in_specs = [
        pl.BlockSpec((_BROWS, DIM), (lambda i, s=s: (s * _GRID + i, 0)))
        for s in range(8)
    ] + [
        pl.BlockSpec((DIM, HID), lambda i: (0, 0)),
        pl.BlockSpec((1, HID), lambda i: (0, 0)),
    ]
    return pl.pallas_call(
        _proj_body,
        grid=(_GRID,),
        in_specs=in_specs,
        out_specs=pl.BlockSpec((_BROWS, 8 * HID), lambda i: (i, 0)),
        out_shape=jax.ShapeDtypeStruct((PACK_ROWS, 8 * HID), jnp.float32),
    )(*([table] * 8), w1, b1row)


def _accum(buf):
    a0 = buf[0]
    a1 = buf[1]
    a2 = buf[2]
    a3 = buf[3]
    for j in range(4, 2 * CHUNK, 4):
        a0 = a0 + buf[j]
        a1 = a1 + buf[j + 1]
        a2 = a2 + buf[j + 2]
        a3 = a3 + buf[j + 3]
    return (a0 + a1) + (a2 + a3)


def _gather_sum(xr, p):
    mesh = plsc.VectorSubcoreMesh(core_axis_name="c", subcore_axis_name="s")

    @functools.partial(
        pl.kernel,
        out_type=jax.ShapeDtypeStruct((B, HID), jnp.float32),
        mesh=mesh,
        scratch_types=[
            pltpu.VMEM((CHUNKS_PER_W, CHUNK), jnp.int32),
            pltpu.VMEM((2 * CHUNK, HID), jnp.float32),
            pltpu.VMEM((2 * CHUNK, HID), jnp.float32),
            pltpu.VMEM((ROWS_PER_W, HID), jnp.float32),
            pltpu.SemaphoreType.DMA,
            pltpu.SemaphoreType.DMA,
            pltpu.SemaphoreType.DMA,
        ],
        compiler_params=pltpu.CompilerParams(use_tc_tiling_on_sc=False),
    )
    def body(x_hbm, p_hbm, out_hbm, idx_v, buf0, buf1, out_v, semi, sem0, sem1):
        wid = lax.axis_index("s") * NC + lax.axis_index("c")
        pltpu.async_copy(x_hbm.at[wid], idx_v, semi).wait()

        def fire_row(r, buf, sem):
            pltpu.async_copy(p_hbm.at[idx_v.at[2 * r]],
                             buf.at[pl.ds(0, CHUNK)], sem)
            pltpu.async_copy(p_hbm.at[idx_v.at[2 * r + 1]],
                             buf.at[pl.ds(CHUNK, CHUNK)], sem)

        def wait_row(buf, sem):
            pltpu.make_async_copy(p_hbm.at[pl.ds(0, 2 * CHUNK)], buf, sem).wait()

        fire_row(0, buf0, sem0)
        fire_row(1, buf1, sem1)

        def step(i, _):
            wait_row(buf0, sem0)
            acc0 = _accum(buf0)
            out_v[2 * i, :] = acc0

            @pl.when(i < ROWS_PER_W // 2 - 1)
            def _():
                fire_row(2 * i + 2, buf0, sem0)

            wait_row(buf1, sem1)
            acc1 = _accum(buf1)
            out_v[2 * i + 1, :] = acc1

            @pl.when(i < ROWS_PER_W // 2 - 1)
            def _():
                fire_row(2 * i + 3, buf1, sem1)

            return 0

        lax.fori_loop(0, ROWS_PER_W // 2, step, 0)
        pltpu.sync_copy(out_v, out_hbm.at[pl.ds(wid * ROWS_PER_W, ROWS_PER_W)])

    return body(xr, p)


def _head_body(s_ref, y_ref, w2t_ref, b2_ref, out_ref):
    h = jnp.maximum(s_ref[...] * (1.0 / L), 0.0)
    w2t = w2t_ref[...]
    l0 = jnp.sum(h * w2t[0][None, :], axis=1) + b2_ref[0, 0]
    l1 = jnp.sum(h * w2t[1][None, :], axis=1) + b2_ref[0, 1]
    m = jnp.maximum(l0, l1)
    lse = m + jnp.log(jnp.exp(l0 - m) + jnp.exp(l1 - m))
    picked = jnp.where(y_ref[...] == 0, l0, l1)
    out_ref[...] = (jnp.sum(lse - picked) * (1.0 / B)).reshape(1, 1)


def _head(s, y, w2t, b2row):
    return pl.pallas_call(
        _head_body,
        out_shape=jax.ShapeDtypeStruct((1, 1), jnp.float32),
    )(s, y, w2t, b2row)


def kernel(x_, y_, table, W1, b1, W2, b2):
    p8 = _project(table, W1, b1.reshape(1, HID))
    p = p8.reshape(VOCAB_PAD, HID)
    xi = x_.astype(jnp.int32)
    xm = 8 * (xi % PACK_ROWS) + xi // PACK_ROWS
    xr = xm.reshape(NW, CHUNKS_PER_W, CHUNK)
    s = _gather_sum(xr, p)
    out = _head(s, y_, W2.T, b2.reshape(1, CLASSES))
    return out[0, 0]


# speedup vs baseline: 25.0126x; 1.3015x over previous
import functools

import jax
import jax.numpy as jnp
from jax import lax
from jax.experimental import pallas as pl
from jax.experimental.pallas import tpu as pltpu
from jax.experimental.pallas import tpu_sc as plsc

VOCAB = 100000
DIM = 128
HID = 16
CLASSES = 2
B = 4096
L = 200

NC = 2
NS = 16
NW = NC * NS
ROWS_PER_W = B // NW
CHUNK = 100
CHUNKS_PER_W = ROWS_PER_W * 2


VOCAB_PAD = 100352
PACK_ROWS = 12544
_GRID = 7
_BROWS = 1792


def _proj_body(*refs):
    t_refs = refs[:8]
    w1_ref, b1_ref, out_ref = refs[8:]
    w1 = w1_ref[...]
    b1 = b1_ref[...]
    parts = [
        jnp.dot(t[...], w1, preferred_element_type=jnp.float32) + b1
        for t in t_refs
    ]
    out_ref[...] = jnp.concatenate(parts, axis=1)


def _project(table, w1, b1row):
    in_specs = [
        pl.BlockSpec((_BROWS, DIM), (lambda i, s=s: (s * _GRID + i, 0)))
        for s in range(8)
    ] + [
        pl.BlockSpec((DIM, HID), lambda i: (0, 0)),
        pl.BlockSpec((1, HID), lambda i: (0, 0)),
    ]
    return pl.pallas_call(
        _proj_body,
        grid=(_GRID,),
        in_specs=in_specs,
        out_specs=pl.BlockSpec((_BROWS, 8 * HID), lambda i: (i, 0)),
        out_shape=jax.ShapeDtypeStruct((PACK_ROWS, 8 * HID), jnp.float32),
    )(*([table] * 8), w1, b1row)


def _accum(buf):
    a0 = buf[0]
    a1 = buf[1]
    a2 = buf[2]
    a3 = buf[3]
    for j in range(4, 2 * CHUNK, 4):
        a0 = a0 + buf[j]
        a1 = a1 + buf[j + 1]
        a2 = a2 + buf[j + 2]
        a3 = a3 + buf[j + 3]
    return (a0 + a1) + (a2 + a3)


def _gather_sum(xr, p):
    mesh = plsc.VectorSubcoreMesh(core_axis_name="c", subcore_axis_name="s")

    NBUF = 4

    @functools.partial(
        pl.kernel,
        out_type=jax.ShapeDtypeStruct((B, HID), jnp.float32),
        mesh=mesh,
        scratch_types=[
            pltpu.VMEM((CHUNKS_PER_W, CHUNK), jnp.int32),
            pltpu.VMEM((NBUF, 2 * CHUNK, HID), jnp.float32),
            pltpu.VMEM((ROWS_PER_W, HID), jnp.float32),
            pltpu.SemaphoreType.DMA,
        ] + [pltpu.SemaphoreType.DMA] * NBUF,
        compiler_params=pltpu.CompilerParams(use_tc_tiling_on_sc=False),
    )
    def body(x_hbm, p_hbm, out_hbm, idx_v, bufs, out_v, semi, *sems):
        wid = lax.axis_index("s") * NC + lax.axis_index("c")
        pltpu.async_copy(x_hbm.at[wid], idx_v, semi).wait()

        def fire_row(r, k):
            pltpu.async_copy(p_hbm.at[idx_v.at[2 * r]],
                             bufs.at[k, pl.ds(0, CHUNK)], sems[k])
            pltpu.async_copy(p_hbm.at[idx_v.at[2 * r + 1]],
                             bufs.at[k, pl.ds(CHUNK, CHUNK)], sems[k])

        def wait_row(k):
            pltpu.make_async_copy(p_hbm.at[pl.ds(0, 2 * CHUNK)],
                                  bufs.at[k], sems[k]).wait()

        for k in range(NBUF):
            fire_row(k, k)

        def step(i, _):
            for k in range(NBUF):
                wait_row(k)
                acc = _accum(bufs.at[k])
                out_v[NBUF * i + k, :] = acc

                @pl.when(i < ROWS_PER_W // NBUF - 1)
                def _():
                    fire_row(NBUF * i + k + NBUF, k)

            return 0

        lax.fori_loop(0, ROWS_PER_W // NBUF, step, 0)
        pltpu.sync_copy(out_v, out_hbm.at[pl.ds(wid * ROWS_PER_W, ROWS_PER_W)])

    return body(xr, p)


def _head_body(s_ref, y_ref, w2t_ref, b2_ref, out_ref):
    h = jnp.maximum(s_ref[...] * (1.0 / L), 0.0)
    w2t = w2t_ref[...]
    l0 = jnp.sum(h * w2t[0][None, :], axis=1) + b2_ref[0, 0]
    l1 = jnp.sum(h * w2t[1][None, :], axis=1) + b2_ref[0, 1]
    m = jnp.maximum(l0, l1)
    lse = m + jnp.log(jnp.exp(l0 - m) + jnp.exp(l1 - m))
    picked = jnp.where(y_ref[...] == 0, l0, l1)
    out_ref[...] = (jnp.sum(lse - picked) * (1.0 / B)).reshape(1, 1)


def _head(s, y, w2t, b2row):
    return pl.pallas_call(
        _head_body,
        out_shape=jax.ShapeDtypeStruct((1, 1), jnp.float32),
    )(s, y, w2t, b2row)


def kernel(x_, y_, table, W1, b1, W2, b2):
    p8 = _project(table, W1, b1.reshape(1, HID))
    p = p8.reshape(VOCAB_PAD, HID)
    xi = x_.astype(jnp.int32)
    xm = 8 * (xi % PACK_ROWS) + xi // PACK_ROWS
    xr = xm.reshape(NW, CHUNKS_PER_W, CHUNK)
    s = _gather_sum(xr, p)
    out = _head(s, y_, W2.T, b2.reshape(1, CLASSES))
    return out[0, 0]


# speedup vs baseline: 25.9376x; 1.0370x over previous
import functools

import jax
import jax.numpy as jnp
from jax import lax
from jax.experimental import pallas as pl
from jax.experimental.pallas import tpu as pltpu
from jax.experimental.pallas import tpu_sc as plsc

VOCAB = 100000
DIM = 128
HID = 16
CLASSES = 2
B = 4096
L = 200

NC = 2
NS = 16
NW = NC * NS
ROWS_PER_W = B // NW

PACK_ROWS = 12544
VOCAB_PAD = 8 * PACK_ROWS
_GRID = 4
_BROWS = PACK_ROWS // _GRID

CHUNK_A = 128
CHUNK_B = 72


def _proj_body(*refs):
    t_refs = refs[:8]
    w1_ref, b1_ref, out_ref = refs[8:]
    w1 = w1_ref[...]
    b1 = b1_ref[...]
    parts = [
        jnp.dot(t[...], w1, preferred_element_type=jnp.float32) + b1
        for t in t_refs
    ]
    out_ref[...] = jnp.concatenate(parts, axis=1)


def _project(table, w1, b1row):
    in_specs = [
        pl.BlockSpec((_BROWS, DIM), (lambda i, s=s: (s * _GRID + i, 0)))
        for s in range(8)
    ] + [
        pl.BlockSpec((DIM, HID), lambda i: (0, 0)),
        pl.BlockSpec((1, HID), lambda i: (0, 0)),
    ]
    return pl.pallas_call(
        _proj_body,
        grid=(_GRID,),
        in_specs=in_specs,
        out_specs=pl.BlockSpec((_BROWS, 8 * HID), lambda i: (i, 0)),
        out_shape=jax.ShapeDtypeStruct((PACK_ROWS, 8 * HID), jnp.float32),
    )(*([table] * 8), w1, b1row)


def _accum(buf):
    a0 = buf[0]
    a1 = buf[1]
    a2 = buf[2]
    a3 = buf[3]
    for j in range(4, L, 4):
        a0 = a0 + buf[j]
        a1 = a1 + buf[j + 1]
        a2 = a2 + buf[j + 2]
        a3 = a3 + buf[j + 3]
    return (a0 + a1) + (a2 + a3)


def _gather_sum(x2, p):
    mesh = plsc.VectorSubcoreMesh(core_axis_name="c", subcore_axis_name="s")
    NBUF = 8

    @functools.partial(
        pl.kernel,
        out_type=jax.ShapeDtypeStruct((B // 8, 8 * HID), jnp.float32),
        mesh=mesh,
        scratch_types=[
            pltpu.VMEM((2 * ROWS_PER_W, 128), jnp.int32),
            pltpu.VMEM((NBUF, L, HID), jnp.float32),
            pltpu.VMEM((ROWS_PER_W // 8, 8 * HID), jnp.float32),
            pltpu.SemaphoreType.DMA,
        ] + [pltpu.SemaphoreType.DMA] * NBUF,
        compiler_params=pltpu.CompilerParams(use_tc_tiling_on_sc=False),
    )
    def body(x_hbm, p_hbm, out_hbm, idx_v, bufs, out_v, semi, *sems):
        wid = lax.axis_index("s") * NC + lax.axis_index("c")
        pltpu.async_copy(
            x_hbm.at[pl.ds(2 * ROWS_PER_W * wid, 2 * ROWS_PER_W)],
            idx_v, semi).wait()

        def fire_row(r, k):
            pltpu.async_copy(p_hbm.at[idx_v.at[2 * r]],
                             bufs.at[k, pl.ds(0, CHUNK_A)], sems[k])
            pltpu.async_copy(p_hbm.at[idx_v.at[2 * r + 1, pl.ds(0, CHUNK_B)]],
                             bufs.at[k, pl.ds(CHUNK_A, CHUNK_B)], sems[k])

        def wait_row(k):
            pltpu.make_async_copy(p_hbm.at[pl.ds(0, L)],
                                  bufs.at[k], sems[k]).wait()

        for k in range(NBUF):
            fire_row(k, k)

        def step(i, _):
            for k in range(NBUF):
                wait_row(k)
                acc = _accum(bufs.at[k])
                out_v[i, pl.ds(HID * k, HID)] = acc

                @pl.when(i < ROWS_PER_W // NBUF - 1)
                def _():
                    fire_row(NBUF * i + k + NBUF, k)

            return 0

        lax.fori_loop(0, ROWS_PER_W // NBUF, step, 0)
        pltpu.sync_copy(
            out_v,
            out_hbm.at[pl.ds(wid * (ROWS_PER_W // 8), ROWS_PER_W // 8)])

    return body(x2, p)


def _head_body(s8_ref, yf_ref, m_ref, b2_ref, out_ref):
    h = jnp.maximum(s8_ref[...] * (1.0 / L), 0.0)
    lg = jnp.dot(h, m_ref[...], preferred_element_type=jnp.float32)
    b2 = b2_ref[...]
    l0 = lg[:, :8] + b2[0, 0]
    l1 = lg[:, 8:] + b2[0, 1]
    mx = jnp.maximum(l0, l1)
    lse = mx + jnp.log(jnp.exp(l0 - mx) + jnp.exp(l1 - mx))
    yf = yf_ref[...]
    picked = l0 + yf * (l1 - l0)
    out_ref[...] = (jnp.sum(lse - picked) * (1.0 / B)).reshape(1, 1)


def _head(s8, yf, m, b2row):
    return pl.pallas_call(
        _head_body,
        out_shape=jax.ShapeDtypeStruct((1, 1), jnp.float32),
    )(s8, yf, m, b2row)


def kernel(x_, y_, table, W1, b1, W2, b2):
    p8 = _project(table, W1, b1.reshape(1, HID))
    p = p8.reshape(VOCAB_PAD, HID)

    xi = x_.astype(jnp.int32)
    xm = 8 * (xi % PACK_ROWS) + xi // PACK_ROWS
    x2 = jnp.concatenate(
        [xm, jnp.zeros((B, 256 - L), jnp.int32)], axis=1).reshape(2 * B, 128)

    s8 = _gather_sum(x2, p)

    eye8 = jnp.eye(8, dtype=jnp.float32)
    m = jnp.concatenate(
        [jnp.kron(eye8, W2[:, 0:1]), jnp.kron(eye8, W2[:, 1:2])], axis=1)
    yf = y_.astype(jnp.float32).reshape(B // 8, 8)

    out = _head(s8, yf, m, b2.reshape(1, CLASSES))
    return out[0, 0]
